# baseline (speedup 1.0000x reference)
import functools

import jax
import jax.numpy as jnp
from jax import lax
from jax.experimental import pallas as pl
from jax.experimental.pallas import tpu as pltpu
from jax.experimental.pallas import tpu_sc as plsc


@functools.lru_cache(maxsize=None)
def _build(B, L, V, D):
    info = plsc.get_sparse_core_info()
    NC, NS = info.num_cores, info.num_subcores
    NW = NC * NS
    LANES = info.num_lanes

    BT = B // NW
    assert BT * NW == B and BT == 128
    assert D % LANES == 0 and D == 64
    DT = D // 8
    G = D // LANES
    PP = 2
    assert L % (2 * PP) == 0

    mesh = plsc.VectorSubcoreMesh(core_axis_name="c", subcore_axis_name="s")

    @functools.partial(
        pl.kernel,
        mesh=mesh,
        out_type=jax.ShapeDtypeStruct((L, DT, NW, 8, BT), jnp.float32),
        compiler_params=pltpu.CompilerParams(use_tc_tiling_on_sc=False,
                                             needs_layout_passes=False),
        scratch_types=[
            pltpu.VMEM((L, BT), jnp.int32),
            pltpu.VMEM((L, D), jnp.float32),
            pltpu.VMEM((PP * BT, D), jnp.float32),
            pltpu.VMEM((PP * BT, D), jnp.float32),
            pltpu.VMEM((PP, DT, 8, BT + 1), jnp.float32),
            pltpu.VMEM((PP, DT, 8, BT + 1), jnp.float32),
            pltpu.SemaphoreType.DMA,
            pltpu.SemaphoreType.DMA,
            pltpu.SemaphoreType.DMA,
            pltpu.SemaphoreType.DMA,
        ],
    )
    def emb_kernel(xt_hbm, tok_hbm, pos_hbm, out_hbm,
                   idx_v, pos_v, st0, st1, ob0, ob1, g0, g1, o0, o1):
        stage = (st0, st1)
        obuf = (ob0, ob1)
        gsem = (g0, g1)
        osem = (o0, o1)
        w = lax.axis_index("s") * NC + lax.axis_index("c")

        pltpu.sync_copy(pos_hbm, pos_v)
        pltpu.sync_copy(xt_hbm.at[:, pl.ds(w * BT, BT)], idx_v)

        def issue_gather(m, b):
            for q in range(PP):
                pltpu.async_copy(tok_hbm.at[idx_v.at[m * PP + q]],
                                 stage[b].at[pl.ds(q * BT, BT)], gsem[b])

        def wait_gather(m, b):
            for q in range(PP):
                pltpu.make_async_copy(tok_hbm.at[idx_v.at[m * PP + q]],
                                      stage[b].at[pl.ds(q * BT, BT)],
                                      gsem[b]).wait()

        def issue_out(m, b):
            pltpu.async_copy(obuf[b].at[:, :, :, pl.ds(0, BT)],
                             out_hbm.at[pl.ds(m * PP, PP), :, w], osem[b])

        def wait_out(m, b):
            pltpu.make_async_copy(obuf[b].at[:, :, :, pl.ds(0, BT)],
                                  out_hbm.at[pl.ds(m * PP, PP), :, w],
                                  osem[b]).wait()

        def compute(m, b):
            lane = lax.iota(jnp.int32, LANES)
            idx_di = lane % 8
            idx_dt = [2 * g + lane // 8 for g in range(G)]
            for q in range(PP):
                pv = [pos_v[m * PP + q, pl.ds(g * LANES, LANES)]
                      for g in range(G)]

                @plsc.parallel_loop(0, BT, unroll=32)
                def _(bi):
                    idx_bi = lane * 0 + bi
                    for g in range(G):
                        v = (stage[b][q * BT + bi, pl.ds(g * LANES, LANES)]
                             + pv[g])
                        plsc.store_scatter(
                            obuf[b].at[q], [idx_dt[g], idx_di, idx_bi], v)

        NM = L // PP
        issue_gather(0, 0)

        def pair_body(i, carry):
            for b in range(2):
                m = i * 2 + b

                @pl.when(m + 1 < NM)
                def _():
                    issue_gather(m + 1, 1 - b)

                wait_gather(m, b)

                @pl.when(m >= 2)
                def _():
                    wait_out(m - 2, b)

                compute(m, b)
                issue_out(m, b)
            return carry

        lax.fori_loop(0, NM // 2, pair_body, 0)
        wait_out(NM - 2, 0)
        wait_out(NM - 1, 1)

    return emb_kernel


def kernel(x, token_table, pos_table):
    B, L = x.shape
    V, D = token_table.shape
    emb = _build(B, L, V, D)
    xt = x.astype(jnp.int32).T
    out5 = emb(xt, token_table, pos_table)
    return out5.transpose(2, 4, 0, 1, 3).reshape(B, L, D)


# speedup vs baseline: 1.2660x; 1.2660x over previous
import functools

import jax
import jax.numpy as jnp
from jax import lax
from jax.experimental import pallas as pl
from jax.experimental.pallas import tpu as pltpu
from jax.experimental.pallas import tpu_sc as plsc


@functools.lru_cache(maxsize=None)
def _build(B, L, V, D):
    info = plsc.get_sparse_core_info()
    NC, NS = info.num_cores, info.num_subcores
    NW = NC * NS
    LANES = info.num_lanes

    BT = B // NW
    assert BT * NW == B and BT == 128
    assert D % LANES == 0 and D == 64
    DT = D // 8
    G = D // LANES
    PP = 2
    NM = L // PP
    assert PP * NM == L and NM % 3 == 1 and NM >= 4

    mesh = plsc.VectorSubcoreMesh(core_axis_name="c", subcore_axis_name="s")

    @functools.partial(
        pl.kernel,
        mesh=mesh,
        out_type=jax.ShapeDtypeStruct((L, DT, NW, 8, BT), jnp.float32),
        compiler_params=pltpu.CompilerParams(use_tc_tiling_on_sc=False,
                                             needs_layout_passes=False),
        scratch_types=[
            pltpu.VMEM((3, PP, BT), jnp.int32),
            pltpu.VMEM((L, D), jnp.float32),
            pltpu.VMEM((PP * BT, D), jnp.float32),
            pltpu.VMEM((PP * BT, D), jnp.float32),
            pltpu.VMEM((PP * BT, D), jnp.float32),
            pltpu.VMEM((PP, DT, 8, BT + 1), jnp.float32),
            pltpu.VMEM((PP, DT, 8, BT + 1), jnp.float32),
            pltpu.VMEM((PP, DT, 8, BT + 1), jnp.float32),
            pltpu.SemaphoreType.DMA,
            pltpu.SemaphoreType.DMA,
            pltpu.SemaphoreType.DMA,
            pltpu.SemaphoreType.DMA,
            pltpu.SemaphoreType.DMA,
            pltpu.SemaphoreType.DMA,
            pltpu.SemaphoreType.DMA,
            pltpu.SemaphoreType.DMA,
            pltpu.SemaphoreType.DMA,
        ],
    )
    def emb_kernel(xt_hbm, tok_hbm, pos_hbm, out_hbm,
                   idx_v, pos_v, st0, st1, st2, ob0, ob1, ob2,
                   i0, i1, i2, g0, g1, g2, o0, o1, o2):
        stage = (st0, st1, st2)
        obuf = (ob0, ob1, ob2)
        isem = (i0, i1, i2)
        gsem = (g0, g1, g2)
        osem = (o0, o1, o2)
        w = lax.axis_index("s") * NC + lax.axis_index("c")

        pltpu.sync_copy(pos_hbm, pos_v)

        def issue_idx(m, j):
            pltpu.async_copy(
                xt_hbm.at[pl.ds(m * PP, PP), pl.ds(w * BT, BT)],
                idx_v.at[j], isem[j])

        def wait_idx(m, j):
            pltpu.make_async_copy(
                xt_hbm.at[pl.ds(m * PP, PP), pl.ds(w * BT, BT)],
                idx_v.at[j], isem[j]).wait()

        def issue_gather(m, j):
            for q in range(PP):
                pltpu.async_copy(tok_hbm.at[idx_v.at[j, q]],
                                 stage[j].at[pl.ds(q * BT, BT)], gsem[j])

        def wait_gather(m, j):
            for q in range(PP):
                pltpu.make_async_copy(tok_hbm.at[idx_v.at[j, q]],
                                      stage[j].at[pl.ds(q * BT, BT)],
                                      gsem[j]).wait()

        def issue_out(m, j):
            pltpu.async_copy(obuf[j].at[:, :, :, pl.ds(0, BT)],
                             out_hbm.at[pl.ds(m * PP, PP), :, w], osem[j])

        def wait_out(m, j):
            pltpu.make_async_copy(obuf[j].at[:, :, :, pl.ds(0, BT)],
                                  out_hbm.at[pl.ds(m * PP, PP), :, w],
                                  osem[j]).wait()

        def compute(m, j):
            lane = lax.iota(jnp.int32, LANES)
            idx_di = lane % 8
            idx_dt = [2 * g + lane // 8 for g in range(G)]
            for q in range(PP):
                pv = [pos_v[m * PP + q, pl.ds(g * LANES, LANES)]
                      for g in range(G)]

                @plsc.parallel_loop(0, BT, unroll=16)
                def _(bi):
                    idx_bi = lane * 0 + bi
                    for g in range(G):
                        v = (stage[j][q * BT + bi, pl.ds(g * LANES, LANES)]
                             + pv[g])
                        plsc.store_scatter(
                            obuf[j].at[q], [idx_dt[g], idx_di, idx_bi], v)

        def body(m, j):
            @pl.when(m + 2 < NM)
            def _():
                issue_idx(m + 2, (j + 2) % 3)

            @pl.when(m + 1 < NM)
            def _():
                wait_idx(m + 1, (j + 1) % 3)
                issue_gather(m + 1, (j + 1) % 3)

            wait_gather(m, j)

            @pl.when(m >= 3)
            def _():
                wait_out(m - 3, j)

            compute(m, j)
            issue_out(m, j)

        issue_idx(0, 0)
        issue_idx(1, 1)
        wait_idx(0, 0)
        issue_gather(0, 0)
        body(0, 0)

        def tri_body(i, carry):
            for b in range(3):
                body(i * 3 + 1 + b, (1 + b) % 3)
            return carry

        lax.fori_loop(0, (NM - 1) // 3, tri_body, 0)
        wait_out(NM - 3, (NM - 3) % 3)
        wait_out(NM - 2, (NM - 2) % 3)
        wait_out(NM - 1, (NM - 1) % 3)

    return emb_kernel


def kernel(x, token_table, pos_table):
    B, L = x.shape
    V, D = token_table.shape
    emb = _build(B, L, V, D)
    xt = x.astype(jnp.int32).T
    out5 = emb(xt, token_table, pos_table)
    return out5.transpose(2, 4, 0, 1, 3).reshape(B, L, D)


# speedup vs baseline: 1.3336x; 1.0534x over previous
import functools

import jax
import jax.numpy as jnp
from jax import lax
from jax.experimental import pallas as pl
from jax.experimental.pallas import tpu as pltpu
from jax.experimental.pallas import tpu_sc as plsc


@functools.lru_cache(maxsize=None)
def _build(B, L, V, D):
    info = plsc.get_sparse_core_info()
    NC, NS = info.num_cores, info.num_subcores
    NW = NC * NS
    LANES = info.num_lanes

    BT = B // NW
    assert BT * NW == B and BT == 128
    assert D % LANES == 0 and D == 64
    DT = D // 8
    G = D // LANES
    PP = 2
    assert L % (2 * PP) == 0

    mesh = plsc.VectorSubcoreMesh(core_axis_name="c", subcore_axis_name="s")

    @functools.partial(
        pl.kernel,
        mesh=mesh,
        out_type=jax.ShapeDtypeStruct((L, DT, NW, 8, BT), jnp.float32),
        compiler_params=pltpu.CompilerParams(use_tc_tiling_on_sc=False,
                                             needs_layout_passes=False),
        scratch_types=[
            pltpu.VMEM((L, BT), jnp.int32),
            pltpu.VMEM((L, D), jnp.float32),
            pltpu.VMEM((PP * BT, D), jnp.float32),
            pltpu.VMEM((PP * BT, D), jnp.float32),
            pltpu.VMEM((PP, DT, 8, BT + 1), jnp.float32),
            pltpu.VMEM((PP, DT, 8, BT + 1), jnp.float32),
            pltpu.SemaphoreType.DMA,
            pltpu.SemaphoreType.DMA,
            pltpu.SemaphoreType.DMA,
            pltpu.SemaphoreType.DMA,
        ],
    )
    def emb_kernel(xt_hbm, tok_hbm, pos_hbm, out_hbm,
                   idx_v, pos_v, st0, st1, ob0, ob1, g0, g1, o0, o1):
        stage = (st0, st1)
        obuf = (ob0, ob1)
        gsem = (g0, g1)
        osem = (o0, o1)
        w = lax.axis_index("s") * NC + lax.axis_index("c")

        pltpu.sync_copy(pos_hbm, pos_v)
        pltpu.sync_copy(xt_hbm.at[:, pl.ds(w * BT, BT)], idx_v)

        def issue_gather(m, b):
            for q in range(PP):
                pltpu.async_copy(tok_hbm.at[idx_v.at[m * PP + q]],
                                 stage[b].at[pl.ds(q * BT, BT)], gsem[b])

        def wait_gather(m, b):
            for q in range(PP):
                pltpu.make_async_copy(tok_hbm.at[idx_v.at[m * PP + q]],
                                      stage[b].at[pl.ds(q * BT, BT)],
                                      gsem[b]).wait()

        def issue_out(m, b):
            pltpu.async_copy(obuf[b].at[:, :, :, pl.ds(0, BT)],
                             out_hbm.at[pl.ds(m * PP, PP), :, w], osem[b])

        def wait_out(m, b):
            pltpu.make_async_copy(obuf[b].at[:, :, :, pl.ds(0, BT)],
                                  out_hbm.at[pl.ds(m * PP, PP), :, w],
                                  osem[b]).wait()

        def compute(m, b):
            lane = lax.iota(jnp.int32, LANES)
            idx_di = lane % 8
            idx_dt = [2 * g + lane // 8 for g in range(G)]
            for q in range(PP):
                pv = [pos_v[m * PP + q, pl.ds(g * LANES, LANES)]
                      for g in range(G)]

                @plsc.parallel_loop(0, BT, unroll=16)
                def _(bi):
                    idx_bi = lane * 0 + bi
                    for g in range(G):
                        v = (stage[b][q * BT + bi, pl.ds(g * LANES, LANES)]
                             + pv[g])
                        plsc.store_scatter(
                            obuf[b].at[q], [idx_dt[g], idx_di, idx_bi], v)

        NM = L // PP
        issue_gather(0, 0)

        def pair_body(i, carry):
            for b in range(2):
                m = i * 2 + b

                @pl.when(m + 1 < NM)
                def _():
                    issue_gather(m + 1, 1 - b)

                wait_gather(m, b)

                @pl.when(m >= 2)
                def _():
                    wait_out(m - 2, b)

                compute(m, b)
                issue_out(m, b)
            return carry

        lax.fori_loop(0, NM // 2, pair_body, 0)
        wait_out(NM - 2, 0)
        wait_out(NM - 1, 1)

    return emb_kernel


def kernel(x, token_table, pos_table):
    B, L = x.shape
    V, D = token_table.shape
    emb = _build(B, L, V, D)
    xt = x.astype(jnp.int32).T
    out5 = emb(xt, token_table, pos_table)
    return out5.transpose(2, 4, 0, 1, 3).reshape(B, L, D)
